# Initial kernel scaffold; baseline (speedup 1.0000x reference)
import functools

import jax
import jax.numpy as jnp
from jax import lax
from jax.experimental import pallas as pl
from jax.experimental.pallas import tpu as pltpu
from jax.experimental.pallas import tpu_sc as plsc

_VOCAB = 100000
_DIM = 128
_B = 4096
_NEG = 20
_K = _NEG + 1
_NW = 32
_IPW = _B // _NW
_CI = 8
_CR = _CI * _K
_NCH = _IPW // _CI
_LANES = 16


def _sc_partials(target_table, context_table, target_idx, ctx_idx_all):
    mesh = plsc.VectorSubcoreMesh(core_axis_name="c", subcore_axis_name="s")

    @functools.partial(
        pl.kernel,
        mesh=mesh,
        out_type=jax.ShapeDtypeStruct((_B * _K, _LANES), jnp.float32),
        scratch_types=[
            pltpu.VMEM((_IPW,), jnp.int32),
            pltpu.VMEM((_IPW * _K,), jnp.int32),
            pltpu.VMEM((_IPW, _DIM), jnp.float32),
            pltpu.VMEM((_CR, _DIM), jnp.float32),
            pltpu.VMEM((_CR, _LANES), jnp.float32),
            pltpu.SemaphoreType.DMA,
        ],
    )
    def k(ttab, ctab, tidx, cidx, out, tixv, cixv, trows, crows, part, sem):
        wid = lax.axis_index("s") * 2 + lax.axis_index("c")
        ibase = wid * _IPW
        rbase = ibase * _K
        pltpu.sync_copy(tidx.at[pl.ds(ibase, _IPW)], tixv)
        pltpu.sync_copy(cidx.at[pl.ds(rbase, _IPW * _K)], cixv)
        pltpu.async_copy(ttab.at[tixv], trows, sem).wait()

        def chunk(ch, carry):
            r0 = ch * _CR
            g1 = pltpu.async_copy(
                ctab.at[cixv.at[pl.ds(r0, 120)]], crows.at[pl.ds(0, 120)], sem)
            g2 = pltpu.async_copy(
                ctab.at[cixv.at[pl.ds(r0 + 120, 48)]], crows.at[pl.ds(120, 48)], sem)
            g1.wait()
            g2.wait()
            for i in range(_CI):
                item = ch * _CI + i
                tv = [trows[item, pl.ds(c * _LANES, _LANES)] for c in range(8)]

                def kl(kk, c2, _i=i, _tv=tv):
                    r = _i * _K + kk
                    acc = _tv[0] * crows[r, pl.ds(0, _LANES)]
                    for c in range(1, 8):
                        acc = acc + _tv[c] * crows[r, pl.ds(c * _LANES, _LANES)]
                    part[r, :] = acc
                    return c2

                lax.fori_loop(0, _K, kl, 0)
            pltpu.sync_copy(part, out.at[pl.ds(rbase + r0, _CR)])
            return carry

        lax.fori_loop(0, _NCH, chunk, 0)

    return k(target_table, context_table, target_idx, ctx_idx_all)


def _tc_loss(partials_mat):
    nrow = _B * _K * _LANES // 128

    def body(x_ref, o_ref):
        x = x_ref[...]
        d = lax.broadcasted_iota(jnp.int32, (128, 8), 0)
        g = lax.broadcasted_iota(jnp.int32, (128, 8), 1)
        m = (d // _LANES == g).astype(jnp.float32)
        s = jnp.dot(x, m, preferred_element_type=jnp.float32)
        j = (lax.broadcasted_iota(jnp.int32, (nrow, 8), 0) * 8
             + lax.broadcasted_iota(jnp.int32, (nrow, 8), 1))
        kk = j % _K
        z = jnp.where(kk == 0, s, -s)
        ll = jnp.minimum(z, 0.0) - jnp.log1p(jnp.exp(-jnp.abs(z)))
        o_ref[0, 0] = -jnp.sum(ll) / _B

    return pl.pallas_call(
        body,
        out_shape=jax.ShapeDtypeStruct((1, 1), jnp.float32),
    )(partials_mat)


def kernel(target_table, context_table, target_idx, context_idx, neg_idx):
    tidx = target_idx.astype(jnp.int32)
    ctx_all = jnp.concatenate(
        [context_idx.astype(jnp.int32)[:, None], neg_idx.astype(jnp.int32)],
        axis=1).reshape(-1)
    partials = _sc_partials(target_table, context_table, tidx, ctx_all)
    pm = partials.reshape(_B * _K * _LANES // 128, 128)
    loss = _tc_loss(pm)
    return loss[0, 0]


# speedup vs baseline: 3.2962x; 3.2962x over previous
import functools

import jax
import jax.numpy as jnp
from jax import lax
from jax.experimental import pallas as pl
from jax.experimental.pallas import tpu as pltpu
from jax.experimental.pallas import tpu_sc as plsc

_VOCAB = 100000
_DIM = 128
_B = 4096
_NEG = 20
_K = _NEG + 1
_NW = 32
_IPW = _B // _NW
_CI = 8
_CR = _CI * _K
_NCH = _IPW // _CI
_LANES = 16


def _sc_partials(target_table, context_table, target_idx, ctx_idx_all):
    mesh = plsc.VectorSubcoreMesh(core_axis_name="c", subcore_axis_name="s")

    @functools.partial(
        pl.kernel,
        mesh=mesh,
        out_type=jax.ShapeDtypeStruct((_B * _K, _LANES), jnp.float32),
        scratch_types=[
            pltpu.VMEM((_IPW,), jnp.int32),
            pltpu.VMEM((_IPW * _K,), jnp.int32),
            pltpu.VMEM((_IPW, _DIM), jnp.float32),
            pltpu.VMEM((_CR, _DIM), jnp.float32),
            pltpu.VMEM((_CR, _LANES), jnp.float32),
            pltpu.SemaphoreType.DMA,
        ],
    )
    def k(ttab, ctab, tidx, cidx, out, tixv, cixv, trows, crows, part, sem):
        wid = lax.axis_index("s") * 2 + lax.axis_index("c")
        ibase = wid * _IPW
        rbase = ibase * _K
        pltpu.sync_copy(tidx.at[pl.ds(ibase, _IPW)], tixv)
        pltpu.sync_copy(cidx.at[pl.ds(rbase, _IPW * _K)], cixv)
        pltpu.async_copy(ttab.at[tixv], trows, sem).wait()

        def chunk(ch, carry):
            r0 = ch * _CR
            g1 = pltpu.async_copy(
                ctab.at[cixv.at[pl.ds(r0, 120)]], crows.at[pl.ds(0, 120)], sem)
            g2 = pltpu.async_copy(
                ctab.at[cixv.at[pl.ds(r0 + 120, 48)]], crows.at[pl.ds(120, 48)], sem)
            g1.wait()
            g2.wait()
            for i in range(_CI):
                item = ch * _CI + i
                tv = [trows[item, pl.ds(c * _LANES, _LANES)] for c in range(8)]

                def kl(kk, c2, _i=i, _tv=tv):
                    r = _i * _K + kk
                    acc = _tv[0] * crows[r, pl.ds(0, _LANES)]
                    for c in range(1, 8):
                        acc = acc + _tv[c] * crows[r, pl.ds(c * _LANES, _LANES)]
                    part[r, :] = acc
                    return c2

                lax.fori_loop(0, _K, kl, 0)
            pltpu.sync_copy(part, out.at[pl.ds(rbase + r0, _CR)])
            return carry

        lax.fori_loop(0, _NCH, chunk, 0)

    return k(target_table, context_table, target_idx, ctx_idx_all)


def _tc_loss(partials_mat):
    nrow = _B * _K * _LANES // 128

    def body(x_ref, o_ref):
        x = x_ref[...]
        d = lax.broadcasted_iota(jnp.int32, (128, 8), 0)
        g = lax.broadcasted_iota(jnp.int32, (128, 8), 1)
        m = (d // _LANES == g).astype(jnp.float32)
        s = jnp.dot(x, m, preferred_element_type=jnp.float32)
        j = (lax.broadcasted_iota(jnp.int32, (nrow, 8), 0) * 8
             + lax.broadcasted_iota(jnp.int32, (nrow, 8), 1))
        kk = j % _K
        z = jnp.where(kk == 0, s, -s)
        ll = jnp.minimum(z, 0.0) - jnp.log1p(jnp.exp(-jnp.abs(z)))
        o_ref[...] = (-jnp.sum(ll) / _B).reshape(1, 1)

    return pl.pallas_call(
        body,
        out_shape=jax.ShapeDtypeStruct((1, 1), jnp.float32),
    )(partials_mat)


def kernel(target_table, context_table, target_idx, context_idx, neg_idx):
    tidx = target_idx.astype(jnp.int32)
    ctx_all = jnp.concatenate(
        [context_idx.astype(jnp.int32)[:, None], neg_idx.astype(jnp.int32)],
        axis=1).reshape(-1)
    partials = _sc_partials(target_table, context_table, tidx, ctx_all)
    pm = partials.reshape(_B * _K * _LANES // 128, 128)
    loss = _tc_loss(pm)
    return loss[0, 0]


# speedup vs baseline: 5.2851x; 1.6034x over previous
import functools

import jax
import jax.numpy as jnp
from jax import lax
from jax.experimental import pallas as pl
from jax.experimental.pallas import tpu as pltpu
from jax.experimental.pallas import tpu_sc as plsc

_VOCAB = 100000
_DIM = 128
_B = 4096
_NEG = 20
_K = _NEG + 1
_NW = 32
_IPW = _B // _NW
_CI = 8
_CR = _CI * _K
_NCH = _IPW // _CI
_LANES = 16
_OROWS = _B * _K * _LANES // 128
_WROWS = _OROWS // _NW
_HROWS = _WROWS // 2


def _sc_partials(target_table, context_table, target_idx, ctx_idx_all):
    mesh = plsc.VectorSubcoreMesh(core_axis_name="c", subcore_axis_name="s")

    @functools.partial(
        pl.kernel,
        mesh=mesh,
        out_type=jax.ShapeDtypeStruct((_OROWS, 128), jnp.float32),
        scratch_types=[
            pltpu.VMEM((_IPW,), jnp.int32),
            pltpu.VMEM((_IPW * _K,), jnp.int32),
            pltpu.VMEM((_IPW, _DIM), jnp.float32),
            pltpu.VMEM((_CR, _DIM), jnp.float32),
            pltpu.VMEM((_CR, _DIM), jnp.float32),
            pltpu.VMEM((_HROWS, 128), jnp.float32),
            pltpu.VMEM((_HROWS, 128), jnp.float32),
            pltpu.SemaphoreType.DMA,
            pltpu.SemaphoreType.DMA,
            pltpu.SemaphoreType.DMA,
            pltpu.SemaphoreType.DMA,
        ],
    )
    def k(ttab, ctab, tidx, cidx, out, tixv, cixv, trows,
          crows_a, crows_b, part0, part1, sem_t, sem_a, sem_b, sem_o):
        wid = lax.axis_index("s") * 2 + lax.axis_index("c")
        ibase = wid * _IPW
        rbase = ibase * _K
        pltpu.sync_copy(tidx.at[pl.ds(ibase, _IPW)], tixv)
        pltpu.sync_copy(cidx.at[pl.ds(rbase, _IPW * _K)], cixv)
        tcopy = pltpu.async_copy(ttab.at[tixv], trows, sem_t)

        crows = (crows_a, crows_b)
        parts = (part0, part1)
        gsem = (sem_a, sem_b)

        def issue(ch):
            b = ch & 1
            r0 = ch * _CR
            c1 = pltpu.async_copy(
                ctab.at[cixv.at[pl.ds(r0, 120)]], crows[b].at[pl.ds(0, 120)],
                gsem[b])
            c2 = pltpu.async_copy(
                ctab.at[cixv.at[pl.ds(r0 + 120, 48)]],
                crows[b].at[pl.ds(120, 48)], gsem[b])
            return (c1, c2)

        pending = issue(0)
        tcopy.wait()
        out_copies = []
        for ch in range(_NCH):
            b = ch & 1
            h = ch // (_NCH // 2)
            nxt = issue(ch + 1) if ch + 1 < _NCH else None
            pending[0].wait()
            pending[1].wait()
            pending = nxt
            cb = crows[b]
            pb = parts[h]
            prow0 = (ch % (_NCH // 2)) * _K

            def item_body(i, carry, _cb=cb, _pb=pb, _prow0=prow0, _ch=ch):
                item = _ch * _CI + i
                tv = [trows[item, pl.ds(c * _LANES, _LANES)] for c in range(8)]

                def kl(kk, c2):
                    f = i * _K + kk
                    acc = tv[0] * _cb[f, pl.ds(0, _LANES)]
                    for c in range(1, 8):
                        acc = acc + tv[c] * _cb[f, pl.ds(c * _LANES, _LANES)]
                    _pb[_prow0 + (f >> 3), pl.ds((f & 7) * _LANES, _LANES)] = acc
                    return c2

                lax.fori_loop(0, _K, kl, 0)
                return carry

            lax.fori_loop(0, _CI, item_body, 0)
            if ch % (_NCH // 2) == (_NCH // 2) - 1:
                out_copies.append(pltpu.async_copy(
                    parts[h], out.at[pl.ds(wid * _WROWS + h * _HROWS, _HROWS)],
                    sem_o))
        for c in out_copies:
            c.wait()

    return k(target_table, context_table, target_idx, ctx_idx_all)


_TCBLK = 1344


def _tc_loss(pm):
    def body(x_ref, o_ref):
        gi = pl.program_id(0)
        x = x_ref[...]
        d = lax.broadcasted_iota(jnp.int32, (128, 8), 0)
        g = lax.broadcasted_iota(jnp.int32, (128, 8), 1)
        m = (d // _LANES == g).astype(jnp.float32)
        s = jnp.dot(x, m, preferred_element_type=jnp.float32)
        j = ((gi * _TCBLK + lax.broadcasted_iota(jnp.int32, (_TCBLK, 8), 0)) * 8
             + lax.broadcasted_iota(jnp.int32, (_TCBLK, 8), 1))
        kk = j % _K
        z = jnp.where(kk == 0, s, -s)
        ll = jnp.minimum(z, 0.0) - jnp.log1p(jnp.exp(-jnp.abs(z)))
        val = (-jnp.sum(ll) / _B).reshape(1, 1)

        @pl.when(gi == 0)
        def _():
            o_ref[...] = val

        @pl.when(gi != 0)
        def _():
            o_ref[...] = o_ref[...] + val

    return pl.pallas_call(
        body,
        grid=(_OROWS // _TCBLK,),
        in_specs=[pl.BlockSpec((_TCBLK, 128), lambda i: (i, 0))],
        out_specs=pl.BlockSpec((1, 1), lambda i: (0, 0)),
        out_shape=jax.ShapeDtypeStruct((1, 1), jnp.float32),
    )(pm)


def kernel(target_table, context_table, target_idx, context_idx, neg_idx):
    tidx = target_idx.astype(jnp.int32)
    ctx_all = jnp.concatenate(
        [context_idx.astype(jnp.int32)[:, None], neg_idx.astype(jnp.int32)],
        axis=1).reshape(-1)
    partials = _sc_partials(target_table, context_table, tidx, ctx_all)
    loss = _tc_loss(partials)
    return loss[0, 0]


# speedup vs baseline: 6.5258x; 1.2348x over previous
import functools

import jax
import jax.numpy as jnp
from jax import lax
from jax.experimental import pallas as pl
from jax.experimental.pallas import tpu as pltpu
from jax.experimental.pallas import tpu_sc as plsc

_VOCAB = 100000
_DIM = 128
_B = 4096
_NEG = 20
_NW = 32
_IPW = _B // _NW
_CI = 8
_CR = _CI * _NEG
_NCH = _IPW // _CI
_LANES = 16
_NROWS = _B * _NEG * _LANES // 128
_OROWS = _NROWS + _B * _LANES // 128
_WROWS = _NROWS // _NW
_HROWS = _WROWS // 2
_PROWS = _IPW * _LANES // 128


def _sc_partials(target_table, context_table, target_idx, context_idx, neg_idx):
    mesh = plsc.VectorSubcoreMesh(core_axis_name="c", subcore_axis_name="s")

    @functools.partial(
        pl.kernel,
        mesh=mesh,
        out_type=jax.ShapeDtypeStruct((_OROWS, 128), jnp.float32),
        scratch_types=[
            pltpu.VMEM((_IPW,), jnp.int32),
            pltpu.VMEM((_IPW,), jnp.int32),
            pltpu.VMEM((_IPW * _NEG,), jnp.int32),
            pltpu.VMEM((_IPW, _DIM), jnp.float32),
            pltpu.VMEM((_IPW, _DIM), jnp.float32),
            pltpu.VMEM((_CR, _DIM), jnp.float32),
            pltpu.VMEM((_CR, _DIM), jnp.float32),
            pltpu.VMEM((_HROWS, 128), jnp.float32),
            pltpu.VMEM((_HROWS, 128), jnp.float32),
            pltpu.VMEM((_PROWS, 128), jnp.float32),
            pltpu.SemaphoreType.DMA,
            pltpu.SemaphoreType.DMA,
            pltpu.SemaphoreType.DMA,
            pltpu.SemaphoreType.DMA,
            pltpu.SemaphoreType.DMA,
        ],
    )
    def k(ttab, ctab, tidx, cidx, nidx, out, tixv, cixv, nixv, trows, cprows,
          nrows_a, nrows_b, part0, part1, ppart,
          sem_t, sem_p, sem_a, sem_b, sem_o):
        wid = lax.axis_index("s") * 2 + lax.axis_index("c")
        ibase = wid * _IPW
        nbase = ibase * _NEG
        pltpu.sync_copy(tidx.at[pl.ds(ibase, _IPW)], tixv)
        pltpu.sync_copy(cidx.at[pl.ds(ibase, _IPW)], cixv)
        pltpu.sync_copy(nidx.at[pl.ds(nbase, _IPW * _NEG)], nixv)
        tcopy = pltpu.async_copy(ttab.at[tixv], trows, sem_t)
        pcopy = pltpu.async_copy(ctab.at[cixv], cprows, sem_p)

        nrows = (nrows_a, nrows_b)
        parts = (part0, part1)
        gsem = (sem_a, sem_b)

        def issue(ch):
            b = ch & 1
            r0 = ch * _CR
            c1 = pltpu.async_copy(
                ctab.at[nixv.at[pl.ds(r0, 96)]], nrows[b].at[pl.ds(0, 96)],
                gsem[b])
            c2 = pltpu.async_copy(
                ctab.at[nixv.at[pl.ds(r0 + 96, 64)]],
                nrows[b].at[pl.ds(96, 64)], gsem[b])
            return (c1, c2)

        pending = issue(0)
        tcopy.wait()
        pcopy.wait()

        @plsc.parallel_loop(0, _IPW, unroll=4)
        def _(i):
            acc = (trows[i, pl.ds(0, _LANES)] * cprows[i, pl.ds(0, _LANES)])
            for c in range(1, 8):
                acc = acc + (trows[i, pl.ds(c * _LANES, _LANES)]
                             * cprows[i, pl.ds(c * _LANES, _LANES)])
            ppart[i >> 3, pl.ds((i & 7) * _LANES, _LANES)] = acc

        out_copies = [pltpu.async_copy(
            ppart, out.at[pl.ds(_NROWS + wid * _PROWS, _PROWS)], sem_o)]

        for ch in range(_NCH):
            b = ch & 1
            h = ch // (_NCH // 2)
            nxt = issue(ch + 1) if ch + 1 < _NCH else None
            pending[0].wait()
            pending[1].wait()
            pending = nxt
            nb = nrows[b]
            pb = parts[h]
            prow0 = (ch % (_NCH // 2)) * (_CR // 8)

            def item_body(i, carry, _nb=nb, _pb=pb, _prow0=prow0, _ch=ch):
                item = _ch * _CI + i
                tvn = [-trows[item, pl.ds(c * _LANES, _LANES)]
                       for c in range(8)]

                @plsc.parallel_loop(0, _NEG, unroll=5)
                def _(kk):
                    f = i * _NEG + kk
                    acc = tvn[0] * _nb[f, pl.ds(0, _LANES)]
                    for c in range(1, 8):
                        acc = acc + tvn[c] * _nb[f, pl.ds(c * _LANES, _LANES)]
                    _pb[_prow0 + (f >> 3),
                        pl.ds((f & 7) * _LANES, _LANES)] = acc

                return carry

            lax.fori_loop(0, _CI, item_body, 0)
            if ch % (_NCH // 2) == (_NCH // 2) - 1:
                out_copies.append(pltpu.async_copy(
                    parts[h], out.at[pl.ds(wid * _WROWS + h * _HROWS, _HROWS)],
                    sem_o))
        for c in out_copies:
            c.wait()

    return k(target_table, context_table, target_idx, context_idx, neg_idx)


_TCBLK = 1344


def _tc_loss(pm):
    def body(x_ref, o_ref):
        gi = pl.program_id(0)
        x = x_ref[...]
        d = lax.broadcasted_iota(jnp.int32, (128, 8), 0)
        g = lax.broadcasted_iota(jnp.int32, (128, 8), 1)
        m = (d // _LANES == g).astype(jnp.float32)
        z = jnp.dot(x, m, preferred_element_type=jnp.float32)
        ll = jnp.minimum(z, 0.0) - jnp.log1p(jnp.exp(-jnp.abs(z)))
        val = (-jnp.sum(ll) / _B).reshape(1, 1)

        @pl.when(gi == 0)
        def _():
            o_ref[...] = val

        @pl.when(gi != 0)
        def _():
            o_ref[...] = o_ref[...] + val

    return pl.pallas_call(
        body,
        grid=(_OROWS // _TCBLK,),
        in_specs=[pl.BlockSpec((_TCBLK, 128), lambda i: (i, 0))],
        out_specs=pl.BlockSpec((1, 1), lambda i: (0, 0)),
        out_shape=jax.ShapeDtypeStruct((1, 1), jnp.float32),
    )(pm)


def kernel(target_table, context_table, target_idx, context_idx, neg_idx):
    tidx = target_idx.astype(jnp.int32)
    cidx = context_idx.astype(jnp.int32)
    nidx = neg_idx.astype(jnp.int32).reshape(-1)
    partials = _sc_partials(target_table, context_table, tidx, cidx, nidx)
    loss = _tc_loss(partials)
    return loss[0, 0]
